# baseline (speedup 1.0000x reference)
import functools

import jax
import jax.numpy as jnp
from jax import lax
from jax.experimental import pallas as pl
from jax.experimental.pallas import tpu as pltpu
from jax.experimental.pallas import tpu_sc as plsc

VOCAB = 1000000
D = 64
N = 4096 * 200

NC = 2
NS = 16
NW = NC * NS
ROWS_PER_W = N // NW
SUB = 128
CHUNK = 512
N_SUB = CHUNK // SUB
N_CHUNKS = ROWS_PER_W // CHUNK

_sc_mesh = plsc.VectorSubcoreMesh(core_axis_name="c", subcore_axis_name="s")


@functools.partial(
    pl.kernel,
    mesh=_sc_mesh,
    out_type=jax.ShapeDtypeStruct((N, D), jnp.float32),
    scratch_types=[
        pltpu.VMEM((N_SUB, SUB), jnp.int32),
        pltpu.VMEM((CHUNK, D), jnp.float32),
        pltpu.SemaphoreType.DMA,
    ],
    compiler_params=pltpu.CompilerParams(use_tc_tiling_on_sc=False),
)
def _sc_gather(idx_hbm, table_hbm, out_hbm, idx_v, buf, sem):
    wid = lax.axis_index("s") * NC + lax.axis_index("c")
    grp0 = wid * (ROWS_PER_W // SUB)
    row0 = wid * ROWS_PER_W

    def body(c, _):
        pltpu.sync_copy(idx_hbm.at[pl.ds(grp0 + c * N_SUB, N_SUB)], idx_v)
        handles = []
        for j in range(N_SUB):
            handles.append(
                pltpu.async_copy(
                    table_hbm.at[idx_v.at[j]],
                    buf.at[pl.ds(j * SUB, SUB)],
                    sem,
                )
            )
        for h in handles:
            h.wait()
        pltpu.sync_copy(buf, out_hbm.at[pl.ds(row0 + c * CHUNK, CHUNK)])
        return ()

    lax.fori_loop(0, N_CHUNKS, body, (), unroll=False)


_TC_BLOCK = 2048
_TC_GRID = N // _TC_BLOCK


def _tc_ce_body(x_ref, t_ref, out_ref):
    x = x_ref[...]
    t = t_ref[0, 0, :]
    m = jnp.max(x, axis=1)
    e = jnp.exp(x - m[:, None])
    s = jnp.sum(e, axis=1)
    logz = jnp.log(s) + m
    onehot = lax.broadcasted_iota(jnp.int32, (_TC_BLOCK, D), 1) == t[:, None]
    picked = jnp.sum(jnp.where(onehot, x, 0.0), axis=1)
    part = jnp.sum(logz - picked)
    i = pl.program_id(0)

    @pl.when(i == 0)
    def _():
        out_ref[0, 0] = 0.0

    out_ref[0, 0] += part

    @pl.when(i == _TC_GRID - 1)
    def _():
        out_ref[0, 0] = out_ref[0, 0] * (1.0 / N)


_tc_ce = pl.pallas_call(
    _tc_ce_body,
    grid=(_TC_GRID,),
    in_specs=[
        pl.BlockSpec((_TC_BLOCK, D), lambda i: (i, 0)),
        pl.BlockSpec((1, 1, _TC_BLOCK), lambda i: (i, 0, 0)),
    ],
    out_specs=pl.BlockSpec(memory_space=pltpu.SMEM),
    out_shape=jax.ShapeDtypeStruct((1, 1), jnp.float32),
)


def kernel(inputs, targets, wte):
    idx = inputs.reshape(N // SUB, SUB).astype(jnp.int32)
    logits2 = _sc_gather(idx, wte)
    tgt = targets.reshape(_TC_GRID, 1, _TC_BLOCK).astype(jnp.int32)
    loss = _tc_ce(logits2, tgt)[0, 0]
    return (logits2, loss)
